# baseline (speedup 1.0000x reference)
import functools
import math

import jax
import jax.numpy as jnp
from jax import lax
from jax.experimental import pallas as pl
from jax.experimental.pallas import tpu as pltpu


def _ln(x, g, b, eps=1e-5):
    mu = jnp.mean(x, axis=-1, keepdims=True)
    xc = x - mu
    v = jnp.mean(xc * xc, axis=-1, keepdims=True)
    return xc * lax.rsqrt(v + eps) * g + b


def _gelu(x):
    c = math.sqrt(2.0 / math.pi)
    return 0.5 * x * (1.0 + jnp.tanh(c * (x + 0.044715 * (x * x * x))))


def _ceil_to(x, m):
    return (x + m - 1) // m * m


def _entity_body(x_ref, wcat_ref, bcat_ref, wp2_ref, bp2_ref, opc_ref,
                 wra_ref, wrb_ref, br1_ref,
                 wd1_ref, bd1_ref, g1_ref, e1_ref,
                 wd2_ref, bd2_ref, g2_ref, e2_ref,
                 st_ref, ph_ref, a_ref, b_ref, dec_ref, *, S):
    x = x_ref[...]
    t = jnp.tanh(jnp.dot(x, wcat_ref[...],
                         preferred_element_type=jnp.float32) + bcat_ref[...])
    re = t[:, :S]
    im = t[:, S:2 * S]
    ph = math.pi * jnp.tanh(
        jnp.dot(t[:, 2 * S:], wp2_ref[...],
                preferred_element_type=jnp.float32) + bp2_ref[...])
    cp = jnp.cos(ph)
    sp = jnp.sin(ph)
    rw = re * cp - im * sp
    iw = re * sp + im * cp
    inv = lax.rsqrt(jnp.sum(rw * rw + iw * iw, axis=-1, keepdims=True) + 1e-12)
    st = jnp.dot(jnp.concatenate([rw * inv, iw * inv], axis=-1), opc_ref[...],
                 preferred_element_type=jnp.float32)
    st_ref[...] = st
    ph_ref[...] = ph

    real = st[:, :S]
    a_ref[...] = jnp.dot(real, wra_ref[...],
                         preferred_element_type=jnp.float32) + br1_ref[...]
    b_ref[...] = jnp.dot(real, wrb_ref[...],
                         preferred_element_type=jnp.float32)

    hd = jnp.dot(real, wd1_ref[...],
                 preferred_element_type=jnp.float32) + bd1_ref[...]
    hd = _gelu(_ln(hd, g1_ref[...], e1_ref[...]))
    y = jnp.dot(hd.astype(jnp.bfloat16), wd2_ref[...],
                preferred_element_type=jnp.float32) + bd2_ref[...]
    dec_ref[...] = _ln(y, g2_ref[...], e2_ref[...])


def _rel_body(a_ref, b_ref, g_ref, be_ref, w2_ref, b2_ref,
              out_ref, sc_ref, cf_ref, *, TI, TJ, Q):
    H = a_ref.shape[1]
    a = a_ref[...]
    b = b_ref[...]
    h = (a[:, None, :] + b[None, :, :]).reshape(TI * TJ, H)
    mu = jnp.mean(h, axis=-1, keepdims=True)
    hc = h - mu
    v = jnp.mean(hc * hc, axis=-1, keepdims=True)
    xn = hc * lax.rsqrt(v + 1e-5) * g_ref[...] + be_ref[...]
    u = _gelu(xn).astype(jnp.bfloat16)
    o = jnp.dot(u, w2_ref[...], preferred_element_type=jnp.float32) + b2_ref[...]
    o3 = o.reshape(TI, TJ, Q)
    out_ref[...] = o3
    sc_ref[...] = jnp.sum(o3, axis=-1)
    nrm = jnp.sqrt(jnp.sum(o3 * o3, axis=-1)) * (1.0 / math.sqrt(Q))
    cf = jnp.minimum(nrm, 1.0)
    ig = pl.program_id(0) * TI + lax.broadcasted_iota(jnp.int32, (TI, TJ), 0)
    jg = pl.program_id(1) * TJ + lax.broadcasted_iota(jnp.int32, (TI, TJ), 1)
    cf_ref[...] = jnp.where(ig == jg, 0.0, cf)


def kernel(entity_emb, w_real, b_real, w_imag, b_imag, w_phase1, b_phase1,
           w_phase2, b_phase2, op_real, op_imag, op_blocks,
           rel_w1, rel_b1, rel_ln_g, rel_ln_b, rel_w2, rel_b2,
           dec_w1, dec_b1, dec_ln1_g, dec_ln1_b, dec_w2, dec_b2,
           dec_ln2_g, dec_ln2_b, normalization):
    x = jnp.asarray(entity_emb, jnp.float32)
    N, E = x.shape
    S = w_phase2.shape[0]
    H = rel_w1.shape[1]
    Q = rel_w2.shape[1]

    w_cat = jnp.concatenate([w_real, w_imag, w_phase1], axis=1)
    b_cat = jnp.concatenate([b_real, b_imag, b_phase1], axis=1)
    opc = op_blocks[0]
    for s in range(1, op_blocks.shape[0]):
        opc = jnp.dot(opc, op_blocks[s], preferred_element_type=jnp.float32)
    wra = rel_w1[:S, :]
    wrb = rel_w1[S:, :]
    wd2_bf = dec_w2.astype(jnp.bfloat16)
    w2_bf = rel_w2.astype(jnp.bfloat16)

    TM = 128 if N % 128 == 0 else 8
    Np = _ceil_to(max(N, 8), TM)
    xp = x if Np == N else jnp.zeros((Np, E), jnp.float32).at[:N].set(x)

    full = lambda i: (0, 0)
    rowb = lambda i: (i, 0)
    st, ph, rel_a, rel_b, dec = pl.pallas_call(
        functools.partial(_entity_body, S=S),
        out_shape=(jax.ShapeDtypeStruct((Np, 2 * S), jnp.float32),
                   jax.ShapeDtypeStruct((Np, S), jnp.float32),
                   jax.ShapeDtypeStruct((Np, H), jnp.float32),
                   jax.ShapeDtypeStruct((Np, H), jnp.float32),
                   jax.ShapeDtypeStruct((Np, E), jnp.float32)),
        grid=(Np // TM,),
        in_specs=[pl.BlockSpec((TM, E), rowb),
                  pl.BlockSpec((E, 3 * S), full),
                  pl.BlockSpec((1, 3 * S), full),
                  pl.BlockSpec((S, S), full),
                  pl.BlockSpec((1, S), full),
                  pl.BlockSpec((2 * S, 2 * S), full),
                  pl.BlockSpec((S, H), full),
                  pl.BlockSpec((S, H), full),
                  pl.BlockSpec((1, H), full),
                  pl.BlockSpec((S, H), full),
                  pl.BlockSpec((1, H), full),
                  pl.BlockSpec((1, H), full),
                  pl.BlockSpec((1, H), full),
                  pl.BlockSpec((H, E), full),
                  pl.BlockSpec((1, E), full),
                  pl.BlockSpec((1, E), full),
                  pl.BlockSpec((1, E), full)],
        out_specs=[pl.BlockSpec((TM, 2 * S), rowb),
                   pl.BlockSpec((TM, S), rowb),
                   pl.BlockSpec((TM, H), rowb),
                   pl.BlockSpec((TM, H), rowb),
                   pl.BlockSpec((TM, E), rowb)],
        compiler_params=pltpu.CompilerParams(dimension_semantics=("parallel",)),
        cost_estimate=pl.CostEstimate(
            flops=Np * 2 * (E * 3 * S + S * S + 4 * S * S + 3 * S * H + H * E),
            transcendentals=Np * (6 * S + H),
            bytes_accessed=4 * Np * (2 * E + 3 * S + 2 * H) + 4 * E * 3 * S
            + 2 * H * E + 4 * 3 * S * H),
    )(xp, w_cat, b_cat, w_phase2, b_phase2, opc, wra, wrb, rel_b1,
      dec_w1, dec_b1, dec_ln1_g, dec_ln1_b, wd2_bf, dec_b2,
      dec_ln2_g, dec_ln2_b)

    TI = 8
    TJ = 512 if N % 512 == 0 else (256 if N % 256 == 0 else
                                   (128 if N % 128 == 0 else N))
    Npr = _ceil_to(max(N, TI), TI)
    Npc = _ceil_to(max(N, TJ), TJ)
    Ap = rel_a[:N] if Npr == N else (
        jnp.zeros((Npr, H), jnp.float32).at[:N].set(rel_a[:N]))
    Bp = rel_b[:N] if Npc == N else (
        jnp.zeros((Npc, H), jnp.float32).at[:N].set(rel_b[:N]))

    rel, sc, cf = pl.pallas_call(
        functools.partial(_rel_body, TI=TI, TJ=TJ, Q=Q),
        out_shape=(jax.ShapeDtypeStruct((Npr, Npc, Q), jnp.float32),
                   jax.ShapeDtypeStruct((Npr, Npc), jnp.float32),
                   jax.ShapeDtypeStruct((Npr, Npc), jnp.float32)),
        grid=(Npr // TI, Npc // TJ),
        in_specs=[pl.BlockSpec((TI, H), lambda i, j: (i, 0)),
                  pl.BlockSpec((TJ, H), lambda i, j: (j, 0)),
                  pl.BlockSpec((1, H), lambda i, j: (0, 0)),
                  pl.BlockSpec((1, H), lambda i, j: (0, 0)),
                  pl.BlockSpec((H, Q), lambda i, j: (0, 0)),
                  pl.BlockSpec((1, Q), lambda i, j: (0, 0))],
        out_specs=[pl.BlockSpec((TI, TJ, Q), lambda i, j: (i, j, 0)),
                   pl.BlockSpec((TI, TJ), lambda i, j: (i, j)),
                   pl.BlockSpec((TI, TJ), lambda i, j: (i, j))],
        compiler_params=pltpu.CompilerParams(
            dimension_semantics=("parallel", "parallel")),
        cost_estimate=pl.CostEstimate(
            flops=Npr * Npc * (2 * H * Q + 16 * H + 4 * Q),
            transcendentals=Npr * Npc * H,
            bytes_accessed=4 * (Npr * H + Npc * H + Npr * Npc * (Q + 2))),
    )(Ap, Bp, rel_ln_g, rel_ln_b, w2_bf, rel_b2)

    enhanced = dec[:N]
    meta = {"relation_states": rel[:N, :N],
            "relation_scores": sc[:N, :N],
            "confidence": cf[:N, :N],
            "quantum_states": {"real": st[:N, :S], "imag": st[:N, S:],
                               "phases": ph[:N]}}
    return enhanced, meta
